# Initial kernel scaffold; baseline (speedup 1.0000x reference)
import jax
import jax.numpy as jnp
from jax.experimental import pallas as pl


def kernel(x, W):
    raise NotImplementedError("write your pallas kernel here")


# speedup vs baseline: 3.2085x; 3.2085x over previous
import functools
import math

import jax
import jax.numpy as jnp
from jax import lax
from jax.experimental import pallas as pl
from jax.experimental.pallas import tpu as pltpu

NUM_EXPERTS = 64
TOKEN_BLOCK = 256
KEY_BIAS = 0x3C000000
AUX_W = 0.01
Z_W = 0.001


def _stage1_body(x_ref, w_ref, s_ref, e_ref, k_ref, psum_ref, zsum_ref):
    i = pl.program_id(0)
    logits = lax.dot_general(
        x_ref[...], w_ref[...], (((1,), (1,)), ((), ())),
        preferred_element_type=jnp.float32)
    m = jnp.max(logits, axis=1, keepdims=True)
    ex = jnp.exp(logits - m)
    ssum = jnp.sum(ex, axis=1, keepdims=True)
    p = ex / ssum
    ei = jnp.argmax(p, axis=1).astype(jnp.int32)
    pm = jnp.max(p, axis=1)
    bits = lax.bitcast_convert_type(pm, jnp.int32)
    key = jnp.maximum(bits - KEY_BIAS, 1)
    onehot = ei[:, None] == lax.broadcasted_iota(jnp.int32, (1, NUM_EXPERTS), 1)
    s_ref[...] = jnp.where(onehot, key[:, None], 0)
    e_ref[0, 0, :] = ei
    k_ref[0, 0, :] = key

    @pl.when(i == 0)
    def _():
        psum_ref[...] = jnp.zeros_like(psum_ref)
        zsum_ref[...] = jnp.zeros_like(zsum_ref)

    psum_ref[...] += jnp.sum(p, axis=0)[None, :]
    lse = m[:, 0] + jnp.log(ssum[:, 0])
    zsum_ref[...] += jnp.sum(lse * lse).reshape(1, 1)


def _stage2_body(capacity, s_ref, t_ref, rem_ref, kept_ref):
    S = s_ref[...]
    cap = jnp.float32(capacity)
    n_e = jnp.sum((S >= 1).astype(jnp.float32), axis=0)[None, :]

    def step(it, t):
        b = 25 - it
        cand = t | (jnp.int32(1) << b)
        cnt = jnp.sum((S >= cand).astype(jnp.float32), axis=0)[None, :]
        return jnp.where(cnt >= cap, cand, t)

    t = lax.fori_loop(0, 26, step, jnp.zeros((1, NUM_EXPERTS), jnp.int32))
    c_gt = jnp.sum((S >= (t + 1)).astype(jnp.float32), axis=0)[None, :]
    rem = jnp.where(t > 0, jnp.int32(capacity) - c_gt.astype(jnp.int32), 0)
    t_ref[...] = t
    rem_ref[...] = rem
    kept_ref[...] = jnp.minimum(n_e, cap).astype(jnp.int32)


def _stage3_body(total_tokens, s_ref, t_ref, rem_ref, kept_ref, psum_ref,
                 zsum_ref, mask_ref, loss_ref, carry_ref):
    i = pl.program_id(0)

    @pl.when(i == 0)
    def _():
        carry_ref[...] = jnp.zeros_like(carry_ref)

    S = s_ref[...]
    t = t_ref[...]
    gt = S > t
    eq = (S == t) & (t > 0)
    eqf = eq.astype(jnp.float32)
    row = lax.broadcasted_iota(jnp.int32, (TOKEN_BLOCK, TOKEN_BLOCK), 0)
    col = lax.broadcasted_iota(jnp.int32, (TOKEN_BLOCK, TOKEN_BLOCK), 1)
    tril = (row > col).astype(jnp.float32)
    excl = lax.dot_general(tril, eqf, (((1,), (0,)), ((), ())),
                           preferred_element_type=jnp.float32)
    rank = carry_ref[...] + excl
    keep_eq = eq & (rank < rem_ref[...].astype(jnp.float32))
    mask_ref[...] = (gt | keep_eq).astype(jnp.float32)
    carry_ref[...] += jnp.sum(eqf, axis=0)[None, :]

    @pl.when(i == pl.num_programs(0) - 1)
    def _():
        n = jnp.float32(total_tokens)
        f = kept_ref[...].astype(jnp.float32) / n
        pmean = psum_ref[...] / n
        aux = AUX_W * jnp.sum(f * pmean) * NUM_EXPERTS
        z = Z_W * zsum_ref[...] / n
        loss_ref[...] = aux + z


def kernel(x, W):
    Bb, Tt, C = x.shape
    E = W.shape[0]
    n = Bb * Tt
    capacity = math.ceil(n / E)
    xr = x.reshape(n, C)
    nblk = n // TOKEN_BLOCK

    S, e_row, k_row, psum, zsum = pl.pallas_call(
        _stage1_body,
        grid=(nblk,),
        in_specs=[
            pl.BlockSpec((TOKEN_BLOCK, C), lambda i: (i, 0)),
            pl.BlockSpec((E, C), lambda i: (0, 0)),
        ],
        out_specs=[
            pl.BlockSpec((TOKEN_BLOCK, E), lambda i: (i, 0)),
            pl.BlockSpec((1, 1, TOKEN_BLOCK), lambda i: (i, 0, 0)),
            pl.BlockSpec((1, 1, TOKEN_BLOCK), lambda i: (i, 0, 0)),
            pl.BlockSpec((1, E), lambda i: (0, 0)),
            pl.BlockSpec((1, 1), lambda i: (0, 0)),
        ],
        out_shape=[
            jax.ShapeDtypeStruct((n, E), jnp.int32),
            jax.ShapeDtypeStruct((nblk, 1, TOKEN_BLOCK), jnp.int32),
            jax.ShapeDtypeStruct((nblk, 1, TOKEN_BLOCK), jnp.int32),
            jax.ShapeDtypeStruct((1, E), jnp.float32),
            jax.ShapeDtypeStruct((1, 1), jnp.float32),
        ],
    )(xr, W)

    t, rem, kept = pl.pallas_call(
        functools.partial(_stage2_body, capacity),
        out_shape=[
            jax.ShapeDtypeStruct((1, E), jnp.int32),
            jax.ShapeDtypeStruct((1, E), jnp.int32),
            jax.ShapeDtypeStruct((1, E), jnp.int32),
        ],
    )(S)

    mask, loss = pl.pallas_call(
        functools.partial(_stage3_body, n),
        grid=(nblk,),
        in_specs=[
            pl.BlockSpec((TOKEN_BLOCK, E), lambda i: (i, 0)),
            pl.BlockSpec((1, E), lambda i: (0, 0)),
            pl.BlockSpec((1, E), lambda i: (0, 0)),
            pl.BlockSpec((1, E), lambda i: (0, 0)),
            pl.BlockSpec((1, E), lambda i: (0, 0)),
            pl.BlockSpec((1, 1), lambda i: (0, 0)),
        ],
        out_specs=[
            pl.BlockSpec((TOKEN_BLOCK, E), lambda i: (i, 0)),
            pl.BlockSpec((1, 1), lambda i: (0, 0)),
        ],
        out_shape=[
            jax.ShapeDtypeStruct((n, E), jnp.float32),
            jax.ShapeDtypeStruct((1, 1), jnp.float32),
        ],
        scratch_shapes=[pltpu.VMEM((1, E), jnp.float32)],
    )(S, t, rem, kept, psum, zsum)

    mask = mask.reshape(Bb, Tt, E)
    return mask, mask, loss.reshape(())


# speedup vs baseline: 3.5433x; 1.1043x over previous
import functools
import math

import jax
import jax.numpy as jnp
from jax import lax
from jax.experimental import pallas as pl
from jax.experimental.pallas import tpu as pltpu
from jax.experimental.pallas import tpu_sc as plsc

NUM_EXPERTS = 64
TOKEN_BLOCK = 256
KEY_BIAS = 0x3C000000
AUX_W = 0.01
Z_W = 0.001


def _stage1_body(x_ref, w_ref, s_ref, e_ref, k_ref, psum_ref, zsum_ref):
    i = pl.program_id(0)
    logits = lax.dot_general(
        x_ref[...], w_ref[...], (((1,), (1,)), ((), ())),
        preferred_element_type=jnp.float32)
    m = jnp.max(logits, axis=1, keepdims=True)
    ex = jnp.exp(logits - m)
    ssum = jnp.sum(ex, axis=1, keepdims=True)
    p = ex / ssum
    ei = jnp.argmax(p, axis=1).astype(jnp.int32)
    pm = jnp.max(p, axis=1)
    bits = lax.bitcast_convert_type(pm, jnp.int32)
    key = jnp.maximum(bits - KEY_BIAS, 1)
    onehot = ei[:, None] == lax.broadcasted_iota(jnp.int32, (1, NUM_EXPERTS), 1)
    s_ref[...] = jnp.where(onehot, key[:, None], 0)
    e_ref[0, 0, :] = ei
    k_ref[0, 0, :] = key

    @pl.when(i == 0)
    def _():
        psum_ref[...] = jnp.zeros_like(psum_ref)
        zsum_ref[...] = jnp.zeros_like(zsum_ref)

    psum_ref[...] += jnp.sum(p, axis=0)[None, :]
    lse = m[:, 0] + jnp.log(ssum[:, 0])
    zsum_ref[...] += jnp.sum(lse * lse).reshape(1, 1)


def _stage2_body(capacity, s_ref, t_ref, rem_ref, kept_ref):
    S = s_ref[...]
    cap = jnp.float32(capacity)
    n_e = jnp.sum((S >= 1).astype(jnp.float32), axis=0)[None, :]

    def step(it, t):
        b = 25 - it
        cand = t | (jnp.int32(1) << b)
        cnt = jnp.sum((S >= cand).astype(jnp.float32), axis=0)[None, :]
        return jnp.where(cnt >= cap, cand, t)

    t = lax.fori_loop(0, 26, step, jnp.zeros((1, NUM_EXPERTS), jnp.int32))
    c_gt = jnp.sum((S >= (t + 1)).astype(jnp.float32), axis=0)[None, :]
    rem = jnp.where(t > 0, jnp.int32(capacity) - c_gt.astype(jnp.int32), 0)
    t_ref[...] = t
    rem_ref[...] = rem
    kept_ref[...] = jnp.minimum(n_e, cap).astype(jnp.int32)


def _sc_count_ge(list_ref, nchunks, u):
    def chunk(j, cnt):
        v = list_ref[pl.ds(j * 16, 16)]
        return cnt + jnp.max(plsc.all_reduce_population_count(v >= u))
    return lax.fori_loop(0, nchunks, chunk, jnp.int32(0))


def _sc_search(list_ref, n, cap):
    nch = (n + 15) // 16
    def bit(i, t):
        cand = t | (jnp.int32(1) << (25 - i))
        c = _sc_count_ge(list_ref, nch, cand)
        return jnp.where(c >= cap, cand, t)
    t = lax.fori_loop(0, 26, bit, jnp.int32(0))
    c_gt = _sc_count_ge(list_ref, nch, t + 1)
    rem = jnp.where(t > 0, cap - c_gt, 0)
    kept = jnp.minimum(n, cap)
    return t, rem, kept


def _stage2_sc_body(capacity, ntok, e_hbm, k_hbm, out_hbm,
                    e_v, k_v, l0_v, l1_v, outbuf_v):
    cap = jnp.int32(capacity)
    wid = lax.axis_index("c") * 16 + lax.axis_index("s")
    ex0 = (wid * 2).astype(jnp.int32)
    ex1 = ex0 + 1
    pltpu.sync_copy(e_hbm, e_v)
    pltpu.sync_copy(k_hbm, k_v)

    def compact(j, carry):
        c0, c1 = carry
        ve = e_v[pl.ds(j * 16, 16)]
        vk = k_v[pl.ds(j * 16, 16)]
        m0 = ve == ex0
        m1 = ve == ex1
        plsc.store_compressed(l0_v.at[pl.ds(c0, 16)], vk, mask=m0)
        plsc.store_compressed(l1_v.at[pl.ds(c1, 16)], vk, mask=m1)
        c0 = c0 + jnp.max(plsc.all_reduce_population_count(m0))
        c1 = c1 + jnp.max(plsc.all_reduce_population_count(m1))
        return c0, c1

    n0, n1 = lax.fori_loop(0, ntok // 16, compact,
                           (jnp.int32(0), jnp.int32(0)))
    zeros = jnp.zeros((16,), jnp.int32)
    l0_v[pl.ds(n0, 16)] = zeros
    l1_v[pl.ds(n1, 16)] = zeros
    t0, rem0, kept0 = _sc_search(l0_v, n0, cap)
    t1, rem1, kept1 = _sc_search(l1_v, n1, cap)

    lane = lax.iota(jnp.int32, 16)
    outv = jnp.where(lane == 0, t0,
           jnp.where(lane == 1, t1,
           jnp.where(lane == 2, rem0,
           jnp.where(lane == 3, rem1,
           jnp.where(lane == 4, kept0,
           jnp.where(lane == 5, kept1, 0))))))
    outbuf_v[...] = outv
    pltpu.sync_copy(outbuf_v, out_hbm.at[wid])


def _stage3_body(total_tokens, s_ref, t_ref, rem_ref, kept_ref, psum_ref,
                 zsum_ref, mask_ref, loss_ref, carry_ref):
    i = pl.program_id(0)

    @pl.when(i == 0)
    def _():
        carry_ref[...] = jnp.zeros_like(carry_ref)

    S = s_ref[...]
    t = t_ref[...]
    gt = S > t
    eq = (S == t) & (t > 0)
    eqf = eq.astype(jnp.float32)
    row = lax.broadcasted_iota(jnp.int32, (TOKEN_BLOCK, TOKEN_BLOCK), 0)
    col = lax.broadcasted_iota(jnp.int32, (TOKEN_BLOCK, TOKEN_BLOCK), 1)
    tril = (row > col).astype(jnp.float32)
    excl = lax.dot_general(tril, eqf, (((1,), (0,)), ((), ())),
                           preferred_element_type=jnp.float32)
    rank = carry_ref[...] + excl
    keep_eq = eq & (rank < rem_ref[...].astype(jnp.float32))
    mask_ref[...] = (gt | keep_eq).astype(jnp.float32)
    carry_ref[...] += jnp.sum(eqf, axis=0)[None, :]

    @pl.when(i == pl.num_programs(0) - 1)
    def _():
        n = jnp.float32(total_tokens)
        f = kept_ref[...].astype(jnp.float32) / n
        pmean = psum_ref[...] / n
        aux = AUX_W * jnp.sum(f * pmean) * NUM_EXPERTS
        z = Z_W * zsum_ref[...] / n
        loss_ref[...] = aux + z


def kernel(x, W):
    Bb, Tt, C = x.shape
    E = W.shape[0]
    n = Bb * Tt
    capacity = math.ceil(n / E)
    xr = x.reshape(n, C)
    nblk = n // TOKEN_BLOCK

    S, e_row, k_row, psum, zsum = pl.pallas_call(
        _stage1_body,
        grid=(nblk,),
        in_specs=[
            pl.BlockSpec((TOKEN_BLOCK, C), lambda i: (i, 0)),
            pl.BlockSpec((E, C), lambda i: (0, 0)),
        ],
        out_specs=[
            pl.BlockSpec((TOKEN_BLOCK, E), lambda i: (i, 0)),
            pl.BlockSpec((1, 1, TOKEN_BLOCK), lambda i: (i, 0, 0)),
            pl.BlockSpec((1, 1, TOKEN_BLOCK), lambda i: (i, 0, 0)),
            pl.BlockSpec((1, E), lambda i: (0, 0)),
            pl.BlockSpec((1, 1), lambda i: (0, 0)),
        ],
        out_shape=[
            jax.ShapeDtypeStruct((n, E), jnp.int32),
            jax.ShapeDtypeStruct((nblk, 1, TOKEN_BLOCK), jnp.int32),
            jax.ShapeDtypeStruct((nblk, 1, TOKEN_BLOCK), jnp.int32),
            jax.ShapeDtypeStruct((1, E), jnp.float32),
            jax.ShapeDtypeStruct((1, 1), jnp.float32),
        ],
    )(xr, W)

    sc_out = pl.kernel(
        functools.partial(_stage2_sc_body, capacity, n),
        out_type=jax.ShapeDtypeStruct((32, 16), jnp.int32),
        mesh=plsc.VectorSubcoreMesh(core_axis_name="c", subcore_axis_name="s"),
        compiler_params=pltpu.CompilerParams(needs_layout_passes=False),
        scratch_types=[
            pltpu.VMEM((n,), jnp.int32),
            pltpu.VMEM((n,), jnp.int32),
            pltpu.VMEM((n + 16,), jnp.int32),
            pltpu.VMEM((n + 16,), jnp.int32),
            pltpu.VMEM((16,), jnp.int32),
        ],
    )(e_row.reshape(n), k_row.reshape(n))
    t = sc_out[:, 0:2].reshape(1, E)
    rem = sc_out[:, 2:4].reshape(1, E)
    kept = sc_out[:, 4:6].reshape(1, E)

    mask, loss = pl.pallas_call(
        functools.partial(_stage3_body, n),
        grid=(nblk,),
        in_specs=[
            pl.BlockSpec((TOKEN_BLOCK, E), lambda i: (i, 0)),
            pl.BlockSpec((1, E), lambda i: (0, 0)),
            pl.BlockSpec((1, E), lambda i: (0, 0)),
            pl.BlockSpec((1, E), lambda i: (0, 0)),
            pl.BlockSpec((1, E), lambda i: (0, 0)),
            pl.BlockSpec((1, 1), lambda i: (0, 0)),
        ],
        out_specs=[
            pl.BlockSpec((TOKEN_BLOCK, E), lambda i: (i, 0)),
            pl.BlockSpec((1, 1), lambda i: (0, 0)),
        ],
        out_shape=[
            jax.ShapeDtypeStruct((n, E), jnp.float32),
            jax.ShapeDtypeStruct((1, 1), jnp.float32),
        ],
        scratch_shapes=[pltpu.VMEM((1, E), jnp.float32)],
    )(S, t, rem, kept, psum, zsum)

    mask = mask.reshape(Bb, Tt, E)
    return mask, mask, loss.reshape(())
